# Initial kernel scaffold; baseline (speedup 1.0000x reference)
import jax
import jax.numpy as jnp
from jax.experimental import pallas as pl


def kernel(x, codebook):
    raise NotImplementedError("write your pallas kernel here")


# speedup vs baseline: 1.8163x; 1.8163x over previous
import jax
import jax.numpy as jnp
from jax.experimental import pallas as pl


def _body(x_ref, cb_ref, out_ref):
    M, D = x_ref.shape
    K = cb_ref.shape[0]
    xv = x_ref[...]
    cb = cb_ref[...]
    x2 = jnp.sum(xv * xv, axis=1, keepdims=True)
    c2 = jnp.sum(cb * cb, axis=1)[None, :]
    xc = jax.lax.dot_general(xv, cb, (((1,), (1,)), ((), ())),
                             preferred_element_type=jnp.float32)
    dist = x2 + c2 - 2.0 * xc
    mins = jnp.min(dist, axis=1, keepdims=True)
    kio = jax.lax.broadcasted_iota(jnp.int32, (M, K), 1)
    idx = jnp.min(jnp.where(dist == mins, kio, K), axis=1, keepdims=True)
    onehot = (kio == idx).astype(jnp.float32)
    out_ref[...] = jax.lax.dot_general(
        onehot, cb, (((1,), (0,)), ((), ())),
        preferred_element_type=jnp.float32,
        precision=jax.lax.Precision.HIGHEST)


def kernel(x, codebook):
    N, C, H, W = x.shape
    K = codebook.shape[0]
    D = H * W
    M = N * C
    x_flat = x.reshape(M, D)
    cb_flat = codebook.reshape(K, D)
    out = pl.pallas_call(
        _body,
        out_shape=jax.ShapeDtypeStruct((M, D), jnp.float32),
    )(x_flat, cb_flat)
    return out.reshape(N, C, H, W)


# speedup vs baseline: 2.1104x; 1.1619x over previous
import jax
import jax.numpy as jnp
from jax.experimental import pallas as pl


def _body(x_ref, cb_ref, out_ref):
    M, D = x_ref.shape
    K = cb_ref.shape[0]
    xv = x_ref[...]
    cb = cb_ref[...]
    x2 = jnp.sum(xv * xv, axis=1, keepdims=True)
    c2 = jnp.sum(cb * cb, axis=1)[None, :]
    xc = jax.lax.dot_general(xv, cb, (((1,), (1,)), ((), ())),
                             preferred_element_type=jnp.float32)
    dist = x2 + c2 - 2.0 * xc
    mins = jnp.min(dist, axis=1, keepdims=True)
    kio = jax.lax.broadcasted_iota(jnp.int32, (M, K), 1)
    idx = jnp.min(jnp.where(dist == mins, kio, K), axis=1, keepdims=True)
    onehot = (kio == idx).astype(jnp.float32)
    out_ref[...] = jax.lax.dot_general(
        onehot, cb, (((1,), (0,)), ((), ())),
        preferred_element_type=jnp.float32)


def kernel(x, codebook):
    N, C, H, W = x.shape
    K = codebook.shape[0]
    D = H * W
    M = N * C
    x_flat = x.reshape(M, D)
    cb_flat = codebook.reshape(K, D)
    out = pl.pallas_call(
        _body,
        out_shape=jax.ShapeDtypeStruct((M, D), jnp.float32),
    )(x_flat, cb_flat)
    return out.reshape(N, C, H, W)
